# baseline (speedup 1.0000x reference)
import jax
import jax.numpy as jnp
from jax import lax
from jax.experimental import pallas as pl
from jax.experimental.pallas import tpu as pltpu
from jax.experimental.pallas import tpu_sc as plsc

_NC = 2
_NS = 16
_NW = _NC * _NS
_SB = 8
_BB = 256
_V = 20


def _sc_body(idx_hbm, tab_hbm, out_hbm, tab_v, idx_a, idx_b, out_a, out_b,
             sem_idx, sem_out):
    c = lax.axis_index("c")
    s = lax.axis_index("s")
    wid = s * _NC + c

    n_s, n_b = idx_hbm.shape
    s_tiles = n_s // _SB
    b_blocks = n_b // _BB
    n_blocks = s_tiles * b_blocks
    per_w = n_blocks // _NW
    k0 = wid * per_w

    pltpu.sync_copy(tab_hbm, tab_v)

    def blk_slices(k):
        s8 = k // b_blocks
        bb = k % b_blocks
        return pl.ds(s8 * _SB, _SB), pl.ds(bb * _BB, _BB)

    ss0, bs0 = blk_slices(k0)
    pltpu.async_copy(idx_hbm.at[ss0, bs0], idx_a, sem_idx)

    def do_block(k, idx_v, out_v, first, last, prefetch=True):
        ss, bs = blk_slices(k)
        pltpu.make_async_copy(idx_hbm.at[ss, bs], idx_v, sem_idx).wait()
        if prefetch:
            kn = jnp.minimum(k + 1, k0 + per_w - 1)
            ssn, bsn = blk_slices(kn)
            nxt = idx_b if idx_v is idx_a else idx_a
            pltpu.async_copy(idx_hbm.at[ssn, bsn], nxt, sem_idx)

        @pl.when(jnp.logical_not(first))
        def _drain():
            pltpu.make_async_copy(out_v, out_hbm.at[:, ss, bs],
                                  sem_out).wait()

        nj = _BB // 16
        lane = lax.iota(jnp.int32, 16)

        @plsc.parallel_loop(0, _SB * nj, unroll=4)
        def _gather_loop(it):
            sr = it // nj
            j = it % nj
            iv = idx_v[sr, pl.ds(j * 16, 16)]
            base = iv * (_V * 16) + lane
            for d in range(_V):
                out_v[d, sr, pl.ds(j * 16, 16)] = plsc.load_gather(
                    tab_v, [base + d * 16])

        cp = pltpu.async_copy(out_v, out_hbm.at[:, ss, bs], sem_out)
        if last:
            cp.wait()

    def step(m, carry):
        do_block(k0 + m * 2, idx_a, out_a, m == 0, False)
        do_block(k0 + m * 2 + 1, idx_b, out_b, m == 0, False)
        return carry

    lax.fori_loop(0, per_w // 2 - 1, step, 0)
    m_last = per_w // 2 - 1
    do_block(k0 + jnp.int32(2 * m_last), idx_a, out_a, m_last == 0, True)
    do_block(k0 + jnp.int32(2 * m_last + 1), idx_b, out_b, False, True,
             prefetch=False)


@jax.jit
def _sc_gather(idx_t, table):
    n_s, n_b = idx_t.shape
    run = pl.kernel(
        _sc_body,
        out_type=jax.ShapeDtypeStruct((_V, n_s, n_b), jnp.float32),
        mesh=plsc.VectorSubcoreMesh(core_axis_name="c", subcore_axis_name="s"),
        scratch_types=[
            pltpu.VMEM((_V * _V * 16,), jnp.float32),
            pltpu.VMEM((_SB, _BB), jnp.int32),
            pltpu.VMEM((_SB, _BB), jnp.int32),
            pltpu.VMEM((_V, _SB, _BB), jnp.float32),
            pltpu.VMEM((_V, _SB, _BB), jnp.float32),
            pltpu.SemaphoreType.DMA,
            pltpu.SemaphoreType.DMA,
        ],
        compiler_params=pltpu.CompilerParams(
            use_tc_tiling_on_sc=True, needs_layout_passes=False),
    )
    return run(idx_t, table)


def kernel(inputs, blosum_probabilities):
    idx_t = jnp.transpose(inputs.astype(jnp.int32), (1, 0))
    rep = jnp.repeat(blosum_probabilities.astype(jnp.float32).reshape(-1), 16)
    g = _sc_gather(idx_t, rep)
    return jnp.transpose(g, (2, 1, 0))


# speedup vs baseline: 1.0619x; 1.0619x over previous
import jax
import jax.numpy as jnp
from jax import lax
from jax.experimental import pallas as pl
from jax.experimental.pallas import tpu as pltpu
from jax.experimental.pallas import tpu_sc as plsc

_NC = 2
_NS = 16
_NW = _NC * _NS
_SB = 8
_BB = 256
_V = 20


def _sc_body(idx_hbm, tab_hbm, out_hbm, tab_v, idx_a, idx_b, out_a, out_b,
             sem_idx, sem_out):
    c = lax.axis_index("c")
    s = lax.axis_index("s")
    wid = s * _NC + c

    n_s, n_b = idx_hbm.shape
    s_tiles = n_s // _SB
    b_blocks = n_b // _BB
    n_blocks = s_tiles * b_blocks
    per_w = n_blocks // _NW
    k0 = wid * per_w

    pltpu.sync_copy(tab_hbm, tab_v)

    def blk_slices(k):
        s8 = k // b_blocks
        bb = k % b_blocks
        return pl.ds(s8 * _SB, _SB), pl.ds(bb * _BB, _BB)

    ss0, bs0 = blk_slices(k0)
    pltpu.async_copy(idx_hbm.at[ss0, bs0], idx_a, sem_idx)

    def do_block(k, idx_v, out_v, first, last, prefetch=True):
        ss, bs = blk_slices(k)
        pltpu.make_async_copy(idx_hbm.at[ss, bs], idx_v, sem_idx).wait()
        if prefetch:
            kn = jnp.minimum(k + 1, k0 + per_w - 1)
            ssn, bsn = blk_slices(kn)
            nxt = idx_b if idx_v is idx_a else idx_a
            pltpu.async_copy(idx_hbm.at[ssn, bsn], nxt, sem_idx)

        @pl.when(jnp.logical_not(first))
        def _drain():
            pltpu.make_async_copy(out_v, out_hbm.at[:, ss, bs],
                                  sem_out).wait()

        nj = _BB // 16
        lane = lax.iota(jnp.int32, 16)

        @plsc.parallel_loop(0, _SB * nj, unroll=2)
        def _gather_loop(it):
            sr = it // nj
            j = it % nj
            iv = idx_v[sr, pl.ds(j * 16, 16)]
            base = iv * (_V * 16) + lane
            for d in range(_V):
                out_v[d, sr, pl.ds(j * 16, 16)] = plsc.load_gather(
                    tab_v, [base + d * 16])

        cp = pltpu.async_copy(out_v, out_hbm.at[:, ss, bs], sem_out)
        if last:
            cp.wait()

    def step(m, carry):
        do_block(k0 + m * 2, idx_a, out_a, m == 0, False)
        do_block(k0 + m * 2 + 1, idx_b, out_b, m == 0, False)
        return carry

    lax.fori_loop(0, per_w // 2 - 1, step, 0)
    m_last = per_w // 2 - 1
    do_block(k0 + jnp.int32(2 * m_last), idx_a, out_a, m_last == 0, True)
    do_block(k0 + jnp.int32(2 * m_last + 1), idx_b, out_b, False, True,
             prefetch=False)


@jax.jit
def _sc_gather(idx_t, table):
    n_s, n_b = idx_t.shape
    run = pl.kernel(
        _sc_body,
        out_type=jax.ShapeDtypeStruct((_V, n_s, n_b), jnp.float32),
        mesh=plsc.VectorSubcoreMesh(core_axis_name="c", subcore_axis_name="s"),
        scratch_types=[
            pltpu.VMEM((_V * _V * 16,), jnp.float32),
            pltpu.VMEM((_SB, _BB), jnp.int32),
            pltpu.VMEM((_SB, _BB), jnp.int32),
            pltpu.VMEM((_V, _SB, _BB), jnp.float32),
            pltpu.VMEM((_V, _SB, _BB), jnp.float32),
            pltpu.SemaphoreType.DMA,
            pltpu.SemaphoreType.DMA,
        ],
        compiler_params=pltpu.CompilerParams(
            use_tc_tiling_on_sc=True, needs_layout_passes=False),
    )
    return run(idx_t, table)


def kernel(inputs, blosum_probabilities):
    idx_t = jnp.transpose(inputs.astype(jnp.int32), (1, 0))
    rep = jnp.repeat(blosum_probabilities.astype(jnp.float32).reshape(-1), 16)
    g = _sc_gather(idx_t, rep)
    return jnp.transpose(g, (2, 1, 0))
